# Initial kernel scaffold; baseline (speedup 1.0000x reference)
import jax
import jax.numpy as jnp
from jax.experimental import pallas as pl


def kernel(x, quantization_choices, W, b):
    raise NotImplementedError("write your pallas kernel here")


# speedup vs baseline: 1.1645x; 1.1645x over previous
import jax
import jax.numpy as jnp
from jax.experimental import pallas as pl
from jax.experimental.pallas import tpu as pltpu
from jax.experimental.pallas import tpu_sc as plsc

_C = 2
_K = 320
_D = 384
_DIN = 768
_CK = _C * _K
_TM = 512
_GW = 128


def _logits_argmax_body(x_ref, wt_ref, b_ref, logits_ref, idx_ref):
    x = x_ref[...]
    logits = jax.lax.dot_general(
        x, wt_ref[...], (((1,), (0,)), ((), ())),
        preferred_element_type=jnp.float32) + b_ref[...]
    logits_ref[...] = logits
    lane = jax.lax.broadcasted_iota(jnp.int32, (_TM, _CK), 1)
    neg = jnp.float32(-jnp.inf)
    in0 = lane < _K
    m0 = jnp.max(jnp.where(in0, logits, neg), axis=1, keepdims=True)
    m1 = jnp.max(jnp.where(in0, neg, logits), axis=1, keepdims=True)
    kmod = jnp.where(in0, lane, lane - _K)
    big = jnp.int32(_CK)
    i0 = jnp.min(jnp.where(in0 & (logits == m0), kmod, big),
                 axis=1, keepdims=True)
    i1 = jnp.min(jnp.where((~in0) & (logits == m1), kmod, big),
                 axis=1, keepdims=True)
    idx_ref[...] = jnp.concatenate([i0, i1 + _K], axis=1)


def _logits_and_indices(xf, wt, b2d):
    t = xf.shape[0]
    return pl.pallas_call(
        _logits_argmax_body,
        grid=(t // _TM,),
        in_specs=[
            pl.BlockSpec((_TM, _DIN), lambda i: (i, 0)),
            pl.BlockSpec((_DIN, _CK), lambda i: (0, 0)),
            pl.BlockSpec((1, _CK), lambda i: (0, 0)),
        ],
        out_specs=[
            pl.BlockSpec((_TM, _CK), lambda i: (i, 0)),
            pl.BlockSpec((_TM, _C), lambda i: (i, 0)),
        ],
        out_shape=[
            jax.ShapeDtypeStruct((t, _CK), jnp.float32),
            jax.ShapeDtypeStruct((t, _C), jnp.int32),
        ],
    )(xf, wt, b2d)


def _sc_gather(qc, idx_flat):
    n = idx_flat.shape[1]
    mesh = plsc.VectorSubcoreMesh(core_axis_name="core",
                                  subcore_axis_name="subcore")

    @pl.kernel(out_type=jax.ShapeDtypeStruct((n, _D), jnp.float32), mesh=mesh)
    def gather_kernel(qc_hbm, i_hbm, o_hbm):
        def body(i_vmem, o_vmem):
            pltpu.sync_copy(qc_hbm.at[i_vmem.at[0]], o_vmem)

        pltpu.emit_pipeline(
            body,
            grid=(n // _GW,),
            in_specs=[pl.BlockSpec((1, _GW), lambda i: (0, i))],
            out_specs=[pl.BlockSpec((_GW, _D), lambda i: (i, 0))],
            core_axis_name=("core", "subcore"),
            dimension_semantics=(pltpu.PARALLEL,),
        )(i_hbm, o_hbm)

    return gather_kernel(qc, idx_flat)


def kernel(x, quantization_choices, W, b):
    B, S, _ = x.shape
    t = B * S
    xf = x.reshape(t, _DIN).astype(jnp.bfloat16)
    wt = W.T.astype(jnp.bfloat16)
    logits, idx = _logits_and_indices(xf, wt, b.reshape(1, _CK))
    rows = _sc_gather(quantization_choices, idx.reshape(1, t * _C))
    q = rows.reshape(B, S, _C * _D)
    return q, logits.reshape(B, S, _C, _K)


# speedup vs baseline: 2.0481x; 1.7588x over previous
import jax
import jax.numpy as jnp
from jax.experimental import pallas as pl
from jax.experimental.pallas import tpu as pltpu
from jax.experimental.pallas import tpu_sc as plsc

_C = 2
_K = 320
_D = 384
_DIN = 768
_CK = _C * _K
_TM = 512
_GW = 128


def _logits_argmax_body(x_ref, wt0_ref, wt1_ref, b_ref, logits_ref, idx_ref):
    x = x_ref[...].astype(jnp.bfloat16)
    dn = (((1,), (0,)), ((), ()))
    l0 = jax.lax.dot_general(x, wt0_ref[...], dn,
                             preferred_element_type=jnp.float32)
    l1 = jax.lax.dot_general(x, wt1_ref[...], dn,
                             preferred_element_type=jnp.float32)
    l0 = l0 + b_ref[0, 0, :][None, :]
    l1 = l1 + b_ref[0, 1, :][None, :]
    logits_ref[:, 0, :] = l0
    logits_ref[:, 1, :] = l1
    lane = jax.lax.broadcasted_iota(jnp.int32, (_TM, _K), 1)
    big = jnp.int32(_CK)
    m0 = jnp.max(l0, axis=1, keepdims=True)
    m1 = jnp.max(l1, axis=1, keepdims=True)
    i0 = jnp.min(jnp.where(l0 == m0, lane, big), axis=1, keepdims=True)
    i1 = jnp.min(jnp.where(l1 == m1, lane, big), axis=1, keepdims=True)
    idx_ref[0:1, :] = i0.T
    idx_ref[1:2, :] = i1.T + _K


def _logits_and_indices(xf, wt0, wt1, b3d):
    t = xf.shape[0]
    return pl.pallas_call(
        _logits_argmax_body,
        grid=(t // _TM,),
        in_specs=[
            pl.BlockSpec((_TM, _DIN), lambda i: (i, 0)),
            pl.BlockSpec((_DIN, _K), lambda i: (0, 0)),
            pl.BlockSpec((_DIN, _K), lambda i: (0, 0)),
            pl.BlockSpec((1, _C, _K), lambda i: (0, 0, 0)),
        ],
        out_specs=[
            pl.BlockSpec((_TM, _C, _K), lambda i: (i, 0, 0)),
            pl.BlockSpec((_C, _TM), lambda i: (0, i)),
        ],
        out_shape=[
            jax.ShapeDtypeStruct((t, _C, _K), jnp.float32),
            jax.ShapeDtypeStruct((_C, t), jnp.int32),
        ],
    )(xf, wt0, wt1, b3d)


def _sc_gather(qc, idx2):
    t = idx2.shape[1]
    mesh = plsc.VectorSubcoreMesh(core_axis_name="core",
                                  subcore_axis_name="subcore")

    @pl.kernel(out_type=jax.ShapeDtypeStruct((t, _C * _D), jnp.float32),
               mesh=mesh)
    def gather_kernel(qc_hbm, i_hbm, o_hbm):
        def body(i_vmem, o_vmem):
            pltpu.sync_copy(qc_hbm.at[i_vmem.at[0]], o_vmem)

        pltpu.emit_pipeline(
            body,
            grid=(t // _GW, _C),
            in_specs=[pl.BlockSpec((1, _GW), lambda i, c: (c, i))],
            out_specs=[pl.BlockSpec((_GW, _D), lambda i, c: (i, c))],
            core_axis_name=("core", "subcore"),
            dimension_semantics=(pltpu.PARALLEL, pltpu.PARALLEL),
        )(i_hbm, o_hbm)

    return gather_kernel(qc, idx2)


def kernel(x, quantization_choices, W, b):
    B, S, _ = x.shape
    t = B * S
    xf = x.reshape(t, _DIN)
    wt = W.T.astype(jnp.bfloat16)
    logits, idx = _logits_and_indices(
        xf, wt[:, :_K], wt[:, _K:], b.reshape(1, _C, _K))
    q = _sc_gather(quantization_choices, idx).reshape(B, S, _C * _D)
    return q, logits.reshape(B, S, _C, _K)


# speedup vs baseline: 2.1294x; 1.0397x over previous
import jax
import jax.numpy as jnp
from jax.experimental import pallas as pl
from jax.experimental.pallas import tpu as pltpu
from jax.experimental.pallas import tpu_sc as plsc

_C = 2
_K = 320
_D = 384
_DIN = 768
_CK = _C * _K
_TM = 1024
_GW = 128


def _logits_argmax_body(x_ref, wt0_ref, wt1_ref, b_ref, logits_ref, idx_ref):
    x = x_ref[...].astype(jnp.bfloat16)
    dn = (((1,), (0,)), ((), ()))
    l0 = jax.lax.dot_general(x, wt0_ref[...], dn,
                             preferred_element_type=jnp.float32)
    l1 = jax.lax.dot_general(x, wt1_ref[...], dn,
                             preferred_element_type=jnp.float32)
    l0 = l0 + b_ref[0, 0, :][None, :]
    l1 = l1 + b_ref[0, 1, :][None, :]
    logits_ref[:, 0, :] = l0
    logits_ref[:, 1, :] = l1
    lane = jax.lax.broadcasted_iota(
        jnp.int32, (_TM, _K), 1).astype(jnp.float32)
    big = jnp.float32(_CK)
    m0 = jnp.max(l0, axis=1, keepdims=True)
    m1 = jnp.max(l1, axis=1, keepdims=True)
    i0 = jnp.min(jnp.where(l0 == m0, lane, big),
                 axis=1, keepdims=True).astype(jnp.int32)
    i1 = jnp.min(jnp.where(l1 == m1, lane, big),
                 axis=1, keepdims=True).astype(jnp.int32)
    idx_ref[0:1, :] = i0.T
    idx_ref[1:2, :] = i1.T + _K


def _logits_and_indices(xf, wt0, wt1, b3d):
    t = xf.shape[0]
    return pl.pallas_call(
        _logits_argmax_body,
        grid=(t // _TM,),
        in_specs=[
            pl.BlockSpec((_TM, _DIN), lambda i: (i, 0)),
            pl.BlockSpec((_DIN, _K), lambda i: (0, 0)),
            pl.BlockSpec((_DIN, _K), lambda i: (0, 0)),
            pl.BlockSpec((1, _C, _K), lambda i: (0, 0, 0)),
        ],
        out_specs=[
            pl.BlockSpec((_TM, _C, _K), lambda i: (i, 0, 0)),
            pl.BlockSpec((_C, _TM), lambda i: (0, i)),
        ],
        out_shape=[
            jax.ShapeDtypeStruct((t, _C, _K), jnp.float32),
            jax.ShapeDtypeStruct((_C, t), jnp.int32),
        ],
    )(xf, wt0, wt1, b3d)


def _sc_gather(qc, idx2):
    t = idx2.shape[1]
    mesh = plsc.VectorSubcoreMesh(core_axis_name="core",
                                  subcore_axis_name="subcore")

    @pl.kernel(out_type=jax.ShapeDtypeStruct((t, _C * _D), jnp.float32),
               mesh=mesh)
    def gather_kernel(qc_hbm, i_hbm, o_hbm):
        def body(i_vmem, o_vmem):
            pltpu.sync_copy(qc_hbm.at[i_vmem.at[0]], o_vmem)

        pltpu.emit_pipeline(
            body,
            grid=(t // _GW, _C),
            in_specs=[pl.BlockSpec((1, _GW), lambda i, c: (c, i))],
            out_specs=[pl.BlockSpec((_GW, _D), lambda i, c: (i, c))],
            core_axis_name=("core", "subcore"),
            dimension_semantics=(pltpu.PARALLEL, pltpu.PARALLEL),
        )(i_hbm, o_hbm)

    return gather_kernel(qc, idx2)


def kernel(x, quantization_choices, W, b):
    B, S, _ = x.shape
    t = B * S
    xf = x.reshape(t, _DIN)
    wt = W.T.astype(jnp.bfloat16)
    logits, idx = _logits_and_indices(
        xf, wt[:, :_K], wt[:, _K:], b.reshape(1, _C, _K))
    q = _sc_gather(quantization_choices, idx).reshape(B, S, _C * _D)
    return q, logits.reshape(B, S, _C, _K)
